# Initial kernel scaffold; baseline (speedup 1.0000x reference)
import functools

import jax
import jax.numpy as jnp
from jax import lax
from jax.experimental import pallas as pl
from jax.experimental.pallas import tpu as pltpu
from jax.experimental.pallas import tpu_sc as plsc

N_SPEC = 512
N_REACT = 32768
N_RATES = 16384
BATCH = 512

NUM_CORES = 2
NUM_SUBCORES = 16
NW = NUM_CORES * NUM_SUBCORES
ROWS_PER_W = BATCH // NW
LANES = 16

CHUNK = 8192
N_CHUNKS = N_REACT // CHUNK
GROUPS = CHUNK // LANES


def _sc_body(y_hbm, rates_hbm, den_hbm, ir0_hbm, ir1_hbm, ik_hbm, ip_hbm,
             sign_hbm, out_hbm,
             y_v, rates_v, acc_v, ir0_v, ir1_v, ik_v, ip_v, sign_v, den_v):
    wid = lax.axis_index("s") * NUM_CORES + lax.axis_index("c")
    row0 = wid * ROWS_PER_W
    pltpu.sync_copy(den_hbm.at[pl.ds(row0, ROWS_PER_W)], den_v)

    def row_body(r, carry):
        row = row0 + r
        pltpu.sync_copy(y_hbm.at[row], y_v)
        pltpu.sync_copy(rates_hbm.at[row], rates_v)

        def zero_body(i, c):
            acc_v[pl.ds(i * LANES, LANES)] = jnp.zeros((LANES,), jnp.float32)
            return c
        lax.fori_loop(0, N_SPEC // LANES, zero_body, 0)

        for c in range(N_CHUNKS):
            pltpu.sync_copy(ir0_hbm.at[pl.ds(c * CHUNK, CHUNK)], ir0_v)
            pltpu.sync_copy(ir1_hbm.at[pl.ds(c * CHUNK, CHUNK)], ir1_v)
            pltpu.sync_copy(ik_hbm.at[pl.ds(c * CHUNK, CHUNK)], ik_v)
            pltpu.sync_copy(ip_hbm.at[pl.ds(c * CHUNK, CHUNK)], ip_v)
            pltpu.sync_copy(sign_hbm.at[pl.ds(c * CHUNK, CHUNK)], sign_v)

            def grp_body(i, cc):
                off = i * LANES
                g0 = ir0_v[pl.ds(off, LANES)]
                g1 = ir1_v[pl.ds(off, LANES)]
                gk = ik_v[pl.ds(off, LANES)]
                gp = ip_v[pl.ds(off, LANES)]
                sg = sign_v[pl.ds(off, LANES)]
                a = plsc.load_gather(y_v, [g0])
                b = plsc.load_gather(y_v, [g1])
                rr = plsc.load_gather(rates_v, [gk])
                t = a * b * rr * sg
                plsc.addupdate_scatter(acc_v, [gp], t)
                return cc
            lax.fori_loop(0, GROUPS, grp_body, 0)

        denv = plsc.load_gather(den_v, [jnp.full((LANES,), r, jnp.int32)])

        def scale_body(i, c2):
            sl = pl.ds(i * LANES, LANES)
            acc_v[sl] = acc_v[sl] * denv
            return c2
        lax.fori_loop(0, N_SPEC // LANES, scale_body, 0)

        pltpu.sync_copy(acc_v, out_hbm.at[row])
        return carry

    lax.fori_loop(0, ROWS_PER_W, row_body, 0)


_sc_kernel = functools.partial(
    pl.kernel,
    out_type=jax.ShapeDtypeStruct((BATCH, N_SPEC), jnp.float32),
    mesh=plsc.VectorSubcoreMesh(core_axis_name="c", subcore_axis_name="s",
                                num_cores=NUM_CORES,
                                num_subcores=NUM_SUBCORES),
    scratch_types=[
        pltpu.VMEM((N_SPEC,), jnp.float32),
        pltpu.VMEM((N_RATES,), jnp.float32),
        pltpu.VMEM((N_SPEC,), jnp.float32),
        pltpu.VMEM((CHUNK,), jnp.int32),
        pltpu.VMEM((CHUNK,), jnp.int32),
        pltpu.VMEM((CHUNK,), jnp.int32),
        pltpu.VMEM((CHUNK,), jnp.int32),
        pltpu.VMEM((CHUNK,), jnp.float32),
        pltpu.VMEM((ROWS_PER_W,), jnp.float32),
    ],
)(_sc_body)


def kernel(y_in, rates, den_norm, inds_r, inds_p, inds_k, rate_sign):
    ir0 = inds_r[:, 0].astype(jnp.int32)
    ir1 = inds_r[:, 1].astype(jnp.int32)
    ik = inds_k.astype(jnp.int32)
    ip = inds_p.astype(jnp.int32)
    den = den_norm.reshape(-1).astype(jnp.float32)
    return _sc_kernel(y_in, rates, den, ir0, ir1, ik, ip,
                      rate_sign.astype(jnp.float32))


# speedup vs baseline: 1.3464x; 1.3464x over previous
import functools

import jax
import jax.numpy as jnp
from jax import lax
from jax.experimental import pallas as pl
from jax.experimental.pallas import tpu as pltpu
from jax.experimental.pallas import tpu_sc as plsc

N_SPEC = 512
N_REACT = 32768
N_RATES = 16384
BATCH = 512

NUM_CORES = 2
NUM_SUBCORES = 16
NW = NUM_CORES * NUM_SUBCORES
ROWS_PER_W = BATCH // NW
LANES = 16

CHUNK = 8192
N_CHUNKS = N_REACT // CHUNK
GROUPS = CHUNK // LANES


def _sc_body(y_hbm, rates_hbm, den_hbm, ir0_hbm, ir1_hbm, ik_hbm, ip_hbm,
             sign_hbm, out_hbm,
             y_v, rates_v, acc_v, ir0_v, ir1_v, ik_v, ip_v, sign_v, den_v):
    wid = lax.axis_index("s") * NUM_CORES + lax.axis_index("c")
    row0 = wid * ROWS_PER_W
    pltpu.sync_copy(den_hbm.at[pl.ds(row0, ROWS_PER_W)], den_v)

    def row_body(r, carry):
        row = row0 + r
        pltpu.sync_copy(y_hbm.at[row], y_v)
        pltpu.sync_copy(rates_hbm.at[row], rates_v)

        def zero_body(i, c):
            acc_v[pl.ds(i * LANES, LANES)] = jnp.zeros((LANES,), jnp.float32)
            return c
        lax.fori_loop(0, N_SPEC // LANES, zero_body, 0)

        for c in range(N_CHUNKS):
            pltpu.sync_copy(ir0_hbm.at[pl.ds(c * CHUNK, CHUNK)], ir0_v)
            pltpu.sync_copy(ir1_hbm.at[pl.ds(c * CHUNK, CHUNK)], ir1_v)
            pltpu.sync_copy(ik_hbm.at[pl.ds(c * CHUNK, CHUNK)], ik_v)
            pltpu.sync_copy(ip_hbm.at[pl.ds(c * CHUNK, CHUNK)], ip_v)
            pltpu.sync_copy(sign_hbm.at[pl.ds(c * CHUNK, CHUNK)], sign_v)

            def grp_body(i, cc):
                off = i * LANES
                g0 = ir0_v[pl.ds(off, LANES)]
                g1 = ir1_v[pl.ds(off, LANES)]
                gk = ik_v[pl.ds(off, LANES)]
                gp = ip_v[pl.ds(off, LANES)]
                sg = sign_v[pl.ds(off, LANES)]
                a = plsc.load_gather(y_v, [g0])
                b = plsc.load_gather(y_v, [g1])
                rr = plsc.load_gather(rates_v, [gk])
                t = a * b * rr * sg
                plsc.addupdate_scatter(acc_v, [gp], t)
                return cc
            lax.fori_loop(0, GROUPS, grp_body, 0)

        denv = plsc.load_gather(den_v, [jnp.full((LANES,), r, jnp.int32)])

        def scale_body(i, c2):
            sl = pl.ds(i * LANES, LANES)
            acc_v[sl] = acc_v[sl] * denv
            return c2
        lax.fori_loop(0, N_SPEC // LANES, scale_body, 0)

        pltpu.sync_copy(acc_v, out_hbm.at[row])
        return carry

    lax.fori_loop(0, ROWS_PER_W, row_body, 0)


_sc_kernel = functools.partial(
    pl.kernel,
    out_type=jax.ShapeDtypeStruct((BATCH, N_SPEC), jnp.float32),
    mesh=plsc.VectorSubcoreMesh(core_axis_name="c", subcore_axis_name="s",
                                num_cores=NUM_CORES,
                                num_subcores=NUM_SUBCORES),
    compiler_params=pltpu.CompilerParams(needs_layout_passes=False),
    scratch_types=[
        pltpu.VMEM((N_SPEC,), jnp.float32),
        pltpu.VMEM((N_RATES,), jnp.float32),
        pltpu.VMEM((N_SPEC,), jnp.float32),
        pltpu.VMEM((CHUNK,), jnp.int32),
        pltpu.VMEM((CHUNK,), jnp.int32),
        pltpu.VMEM((CHUNK,), jnp.int32),
        pltpu.VMEM((CHUNK,), jnp.int32),
        pltpu.VMEM((CHUNK,), jnp.float32),
        pltpu.VMEM((ROWS_PER_W,), jnp.float32),
    ],
)(_sc_body)


def kernel(y_in, rates, den_norm, inds_r, inds_p, inds_k, rate_sign):
    ir0 = inds_r[:, 0].astype(jnp.int32)
    ir1 = inds_r[:, 1].astype(jnp.int32)
    ik = inds_k.astype(jnp.int32)
    ip = inds_p.astype(jnp.int32)
    den = den_norm.reshape(-1).astype(jnp.float32)
    return _sc_kernel(y_in, rates, den, ir0, ir1, ik, ip,
                      rate_sign.astype(jnp.float32))


# speedup vs baseline: 5.6482x; 4.1952x over previous
import functools

import jax
import jax.numpy as jnp
from jax import lax
from jax.experimental import pallas as pl
from jax.experimental.pallas import tpu as pltpu
from jax.experimental.pallas import tpu_sc as plsc

N_SPEC = 512
N_REACT = 32768
N_RATES = 16384
BATCH = 512

NUM_CORES = 2
NUM_SUBCORES = 16
NW = NUM_CORES * NUM_SUBCORES
ROWS_PER_W = BATCH // NW
LANES = 16
ROW_BLK = 2
N_BLKS = ROWS_PER_W // ROW_BLK


def _sc_body(y_hbm, rates_hbm, den_hbm, pa_hbm, pb_hbm, out_hbm,
             pa_v, pb_v, y0_v, y1_v, r0_v, r1_v, acc0_v, acc1_v, den_v):
    wid = lax.axis_index("s") * NUM_CORES + lax.axis_index("c")
    row0 = wid * ROWS_PER_W
    pltpu.sync_copy(pa_hbm, pa_v)
    pltpu.sync_copy(pb_hbm, pb_v)
    pltpu.sync_copy(den_hbm.at[pl.ds(row0, ROWS_PER_W)], den_v)

    def blk_body(rb, carry):
        row = row0 + rb * ROW_BLK
        pltpu.sync_copy(y_hbm.at[row], y0_v)
        pltpu.sync_copy(y_hbm.at[row + 1], y1_v)
        pltpu.sync_copy(rates_hbm.at[row], r0_v)
        pltpu.sync_copy(rates_hbm.at[row + 1], r1_v)

        def zero_body(i, c):
            acc0_v[pl.ds(i * LANES, LANES)] = jnp.zeros((LANES,), jnp.float32)
            acc1_v[pl.ds(i * LANES, LANES)] = jnp.zeros((LANES,), jnp.float32)
            return c
        lax.fori_loop(0, (2 * N_SPEC) // LANES, zero_body, 0)

        @plsc.parallel_loop(0, N_REACT, step=LANES, unroll=4)
        def grp_body(i):
            pa = pa_v[pl.ds(i, LANES)]
            pb = pb_v[pl.ds(i, LANES)]
            i0 = pa & 511
            i1 = pa >> 9
            kk = pb & 16383
            px = pb >> 14
            for y_v, r_v, acc_v in ((y0_v, r0_v, acc0_v),
                                    (y1_v, r1_v, acc1_v)):
                ya = plsc.load_gather(y_v, [i0])
                yb = plsc.load_gather(y_v, [i1])
                rr = plsc.load_gather(r_v, [kk])
                plsc.addupdate_scatter(acc_v, [px], ya * yb * rr)

        for g, acc_v in ((0, acc0_v), (1, acc1_v)):
            denv = plsc.load_gather(
                den_v, [jnp.full((LANES,), rb * ROW_BLK + g, jnp.int32)])

            def fin_body(i, c, acc_v=acc_v, denv=denv):
                sl = pl.ds(i * LANES, LANES)
                neg = acc_v[pl.ds(i * LANES + N_SPEC, LANES)]
                acc_v[sl] = (acc_v[sl] - neg) * denv
                return c
            lax.fori_loop(0, N_SPEC // LANES, fin_body, 0)
            pltpu.sync_copy(acc_v.at[pl.ds(0, N_SPEC)], out_hbm.at[row + g])
        return carry

    lax.fori_loop(0, N_BLKS, blk_body, 0)


_sc_kernel = functools.partial(
    pl.kernel,
    out_type=jax.ShapeDtypeStruct((BATCH, N_SPEC), jnp.float32),
    mesh=plsc.VectorSubcoreMesh(core_axis_name="c", subcore_axis_name="s",
                                num_cores=NUM_CORES,
                                num_subcores=NUM_SUBCORES),
    compiler_params=pltpu.CompilerParams(needs_layout_passes=False),
    scratch_types=[
        pltpu.VMEM((N_REACT,), jnp.int32),
        pltpu.VMEM((N_REACT,), jnp.int32),
        pltpu.VMEM((N_SPEC,), jnp.float32),
        pltpu.VMEM((N_SPEC,), jnp.float32),
        pltpu.VMEM((N_RATES,), jnp.float32),
        pltpu.VMEM((N_RATES,), jnp.float32),
        pltpu.VMEM((2 * N_SPEC,), jnp.float32),
        pltpu.VMEM((2 * N_SPEC,), jnp.float32),
        pltpu.VMEM((ROWS_PER_W,), jnp.float32),
    ],
)(_sc_body)


def kernel(y_in, rates, den_norm, inds_r, inds_p, inds_k, rate_sign):
    ir0 = inds_r[:, 0].astype(jnp.int32)
    ir1 = inds_r[:, 1].astype(jnp.int32)
    ik = inds_k.astype(jnp.int32)
    ipx = inds_p.astype(jnp.int32) + jnp.where(rate_sign < 0, N_SPEC, 0)
    packed_a = ir0 | (ir1 << 9)
    packed_b = ik | (ipx << 14)
    den = den_norm.reshape(-1).astype(jnp.float32)
    return _sc_kernel(y_in, rates, den, packed_a, packed_b)


# speedup vs baseline: 6.0757x; 1.0757x over previous
import functools

import jax
import jax.numpy as jnp
from jax import lax
from jax.experimental import pallas as pl
from jax.experimental.pallas import tpu as pltpu
from jax.experimental.pallas import tpu_sc as plsc

N_SPEC = 512
N_REACT = 32768
N_RATES = 16384
BATCH = 512

NUM_CORES = 2
NUM_SUBCORES = 16
NW = NUM_CORES * NUM_SUBCORES
ROWS_PER_W = BATCH // NW
LANES = 16
ROW_BLK = 2
N_BLKS = ROWS_PER_W // ROW_BLK


def _sc_body(y_hbm, rates_hbm, den_hbm, pa_hbm, pb_hbm, out_hbm,
             pa_v, pb_v, y0_v, y1_v, r0_v, r1_v, acc0_v, acc1_v, den_v,
             dma_sem):
    wid = lax.axis_index("s") * NUM_CORES + lax.axis_index("c")
    row0 = wid * ROWS_PER_W
    pltpu.sync_copy(pa_hbm, pa_v)
    pltpu.sync_copy(pb_hbm, pb_v)
    pltpu.sync_copy(den_hbm.at[pl.ds(row0, ROWS_PER_W)], den_v)

    def blk_body(rb, carry):
        row = row0 + rb * ROW_BLK
        copies = [pltpu.async_copy(y_hbm.at[row], y0_v, dma_sem),
                  pltpu.async_copy(y_hbm.at[row + 1], y1_v, dma_sem),
                  pltpu.async_copy(rates_hbm.at[row], r0_v, dma_sem),
                  pltpu.async_copy(rates_hbm.at[row + 1], r1_v, dma_sem)]

        def zero_body(i, c):
            acc0_v[pl.ds(i * LANES, LANES)] = jnp.zeros((LANES,), jnp.float32)
            acc1_v[pl.ds(i * LANES, LANES)] = jnp.zeros((LANES,), jnp.float32)
            return c
        lax.fori_loop(0, (2 * N_SPEC) // LANES, zero_body, 0)
        for h in copies:
            h.wait()

        @plsc.parallel_loop(0, N_REACT, step=LANES, unroll=8)
        def grp_body(i):
            pa = pa_v[pl.ds(i, LANES)]
            pb = pb_v[pl.ds(i, LANES)]
            i0 = pa & 511
            i1 = pa >> 9
            kk = pb & 16383
            px = pb >> 14
            for y_v, r_v, acc_v in ((y0_v, r0_v, acc0_v),
                                    (y1_v, r1_v, acc1_v)):
                ya = plsc.load_gather(y_v, [i0])
                yb = plsc.load_gather(y_v, [i1])
                rr = plsc.load_gather(r_v, [kk])
                plsc.addupdate_scatter(acc_v, [px], ya * yb * rr)

        for g, acc_v in ((0, acc0_v), (1, acc1_v)):
            denv = plsc.load_gather(
                den_v, [jnp.full((LANES,), rb * ROW_BLK + g, jnp.int32)])

            def fin_body(i, c, acc_v=acc_v, denv=denv):
                sl = pl.ds(i * LANES, LANES)
                neg = acc_v[pl.ds(i * LANES + N_SPEC, LANES)]
                acc_v[sl] = (acc_v[sl] - neg) * denv
                return c
            lax.fori_loop(0, N_SPEC // LANES, fin_body, 0)
            pltpu.sync_copy(acc_v.at[pl.ds(0, N_SPEC)], out_hbm.at[row + g])
        return carry

    lax.fori_loop(0, N_BLKS, blk_body, 0)


_sc_kernel = functools.partial(
    pl.kernel,
    out_type=jax.ShapeDtypeStruct((BATCH, N_SPEC), jnp.float32),
    mesh=plsc.VectorSubcoreMesh(core_axis_name="c", subcore_axis_name="s",
                                num_cores=NUM_CORES,
                                num_subcores=NUM_SUBCORES),
    compiler_params=pltpu.CompilerParams(needs_layout_passes=False),
    scratch_types=[
        pltpu.VMEM((N_REACT,), jnp.int32),
        pltpu.VMEM((N_REACT,), jnp.int32),
        pltpu.VMEM((N_SPEC,), jnp.float32),
        pltpu.VMEM((N_SPEC,), jnp.float32),
        pltpu.VMEM((N_RATES,), jnp.float32),
        pltpu.VMEM((N_RATES,), jnp.float32),
        pltpu.VMEM((2 * N_SPEC,), jnp.float32),
        pltpu.VMEM((2 * N_SPEC,), jnp.float32),
        pltpu.VMEM((ROWS_PER_W,), jnp.float32),
        pltpu.SemaphoreType.DMA,
    ],
)(_sc_body)


def kernel(y_in, rates, den_norm, inds_r, inds_p, inds_k, rate_sign):
    ir0 = inds_r[:, 0].astype(jnp.int32)
    ir1 = inds_r[:, 1].astype(jnp.int32)
    ik = inds_k.astype(jnp.int32)
    ipx = inds_p.astype(jnp.int32) + jnp.where(rate_sign < 0, N_SPEC, 0)
    packed_a = ir0 | (ir1 << 9)
    packed_b = ik | (ipx << 14)
    den = den_norm.reshape(-1).astype(jnp.float32)
    return _sc_kernel(y_in, rates, den, packed_a, packed_b)
